# Initial kernel scaffold; baseline (speedup 1.0000x reference)
import jax
import jax.numpy as jnp
from jax.experimental import pallas as pl


def kernel(positions, hop_distances):
    raise NotImplementedError("write your pallas kernel here")


# speedup vs baseline: 6.2560x; 6.2560x over previous
import functools

import jax
import jax.numpy as jnp
import numpy as np
from jax.experimental import pallas as pl

N = 1024
TOP_K = 10
KK1 = TOP_K // 2 + 1
KK2 = TOP_K + 1
RBF_START = 0.0
RBF_STOP = 20.0
RBF_NUM = 64
RBF_WIDTH = 1.0

BLOCK_ROWS = 128
NUM_BLOCKS = N // BLOCK_ROWS

_BIG = 1e9


def _graph_kernel(pos_ref, post_ref, hop_ref, rnd_ref,
                  dst_ref, dist_ref, vec_ref, rbf_ref, rot_ref):
    pid = pl.program_id(0)
    row0 = pid * BLOCK_ROWS

    post = post_ref[...]
    hop = hop_ref[...]
    posb = pos_ref[pl.ds(row0, BLOCK_ROWS), :]

    col = jax.lax.broadcasted_iota(jnp.int32, (BLOCK_ROWS, N), 1)
    rowi = jax.lax.broadcasted_iota(jnp.int32, (BLOCK_ROWS, N), 0) + row0

    bd = jnp.where(hop == 0.0, jnp.float32(999.0), hop)
    work = bd
    m1 = jnp.zeros((BLOCK_ROWS, N), dtype=jnp.bool_)
    for _ in range(KK1):
        mn = jnp.min(work, axis=1, keepdims=True)
        jmin = jnp.min(jnp.where(work == mn, col, N), axis=1, keepdims=True)
        sel = col == jmin
        m1 = jnp.logical_or(m1, sel)
        work = jnp.where(sel, _BIG, work)
    dmask = jnp.logical_and(m1, hop > 0.0)

    px, py, pz = post[0:1, :], post[1:2, :], post[2:3, :]
    pos = pos_ref[...]
    dots = jax.lax.dot_general(
        posb, pos, (((1,), (1,)), ((), ())),
        preferred_element_type=jnp.float32)
    sa = (posb[:, 0:1] * posb[:, 0:1]
          + posb[:, 1:2] * posb[:, 1:2]
          + posb[:, 2:3] * posb[:, 2:3])
    sb = px * px + py * py + pz * pz
    d2 = sa + sb - 2.0 * dots
    R = jnp.sqrt(jnp.maximum(d2, 0.0))

    work = jnp.where(dmask, 0.0, R)
    m2 = jnp.zeros((BLOCK_ROWS, N), dtype=jnp.bool_)
    for _ in range(KK2):
        mn = jnp.min(work, axis=1, keepdims=True)
        jmin = jnp.min(jnp.where(work == mn, col, N), axis=1, keepdims=True)
        sel = col == jmin
        m2 = jnp.logical_or(m2, sel)
        work = jnp.where(sel, _BIG, work)
    emask = jnp.logical_and(m2, col != rowi)

    widx = jnp.where(emask, col, N)
    dst_cols, dist_cols, vx_cols, vy_cols, vz_cols = [], [], [], [], []
    for _ in range(TOP_K):
        jm = jnp.min(widx, axis=1)
        onehot = col == jm[:, None]
        dst_cols.append(jm[:, None])
        dist_cols.append(
            jnp.sum(jnp.where(onehot, R, 0.0), axis=1)[:, None])
        gx = jnp.sum(jnp.where(onehot, px, 0.0), axis=1)[:, None]
        gy = jnp.sum(jnp.where(onehot, py, 0.0), axis=1)[:, None]
        gz = jnp.sum(jnp.where(onehot, pz, 0.0), axis=1)[:, None]
        vx_cols.append(gx - posb[:, 0:1])
        vy_cols.append(gy - posb[:, 1:2])
        vz_cols.append(gz - posb[:, 2:3])
        widx = jnp.where(onehot, N, widx)

    dst = jnp.concatenate(dst_cols, axis=1)
    dist = jnp.concatenate(dist_cols, axis=1)
    vx = jnp.concatenate(vx_cols, axis=1)
    vy = jnp.concatenate(vy_cols, axis=1)
    vz = jnp.concatenate(vz_cols, axis=1)
    vec = jnp.concatenate(
        [vx[:, :, None], vy[:, :, None], vz[:, :, None]], axis=2)

    dst_ref[...] = dst
    dist_ref[...] = dist
    vec_ref[...] = vec

    step = (RBF_STOP - RBF_START) / (RBF_NUM - 1)
    offset = (RBF_START
              + step * jax.lax.broadcasted_iota(
                  jnp.int32, (1, 1, RBF_NUM), 2).astype(jnp.float32))
    coeff = -0.5 / (RBF_WIDTH * step) ** 2
    rbf_ref[...] = jnp.exp(coeff * (dist[..., None] - offset) ** 2)

    dx, dy, dz = vx, vy, vz
    dd = jnp.sqrt(jnp.maximum(dx * dx + dy * dy + dz * dz, 1e-12))
    nxx, nxy, nxz = dx / dd, dy / dd, dz / dd

    rnd = rnd_ref[...]
    rx, ry, rz = rnd[:, :, 0], rnd[:, :, 1], rnd[:, :, 2]
    rn = jnp.sqrt(rx * rx + ry * ry + rz * rz)
    ex, ey, ez = rx / rn, ry / rn, rz / rn

    vd_a = jnp.abs(ex * nxx + ey * nxy + ez * nxz)
    vd_b = jnp.abs(-ey * nxx + ex * nxy + ez * nxz)
    take_b = vd_a > vd_b
    ex2 = jnp.where(take_b, -ey, ex)
    ey2 = jnp.where(take_b, ex, ey)
    ez2 = jnp.where(take_b, ez, ez)
    vd = jnp.minimum(vd_a, vd_b)
    vd_c = jnp.abs(ex * nxx - ez * nxy + ey * nxz)
    take_c = vd > vd_c
    ex3 = jnp.where(take_c, ex, ex2)
    ey3 = jnp.where(take_c, -ez, ey2)
    ez3 = jnp.where(take_c, ey, ez2)

    zx = nxy * ez3 - nxz * ey3
    zy = nxz * ex3 - nxx * ez3
    zz = nxx * ey3 - nxy * ex3
    zn = jnp.sqrt(zx * zx + zy * zy + zz * zz)
    zx, zy, zz = zx / zn, zy / zn, zz / zn
    yx = nxy * zz - nxz * zy
    yy = nxz * zx - nxx * zz
    yz = nxx * zy - nxy * zx
    yn = jnp.sqrt(yx * yx + yy * yy + yz * yz)
    yx, yy, yz = yx / yn, yy / yn, yz / yn

    comps = [zx, nxx, -yx, zy, nxy, -yy, zz, nxz, -yz]
    rot_ref[...] = jnp.concatenate([c[:, :, None] for c in comps], axis=2)


@functools.partial(jax.jit, static_argnums=())
def kernel(positions, hop_distances):
    rnd = jax.random.uniform(jax.random.key(42), (N * TOP_K, 3),
                             dtype=jnp.float32) - 0.5
    rnd = rnd.reshape(N, TOP_K, 3)

    grid = (NUM_BLOCKS,)
    out = pl.pallas_call(
        _graph_kernel,
        grid=grid,
        in_specs=[
            pl.BlockSpec((N, 3), lambda i: (0, 0)),
            pl.BlockSpec((3, N), lambda i: (0, 0)),
            pl.BlockSpec((BLOCK_ROWS, N), lambda i: (i, 0)),
            pl.BlockSpec((BLOCK_ROWS, TOP_K, 3), lambda i: (i, 0, 0)),
        ],
        out_specs=[
            pl.BlockSpec((BLOCK_ROWS, TOP_K), lambda i: (i, 0)),
            pl.BlockSpec((BLOCK_ROWS, TOP_K), lambda i: (i, 0)),
            pl.BlockSpec((BLOCK_ROWS, TOP_K, 3), lambda i: (i, 0, 0)),
            pl.BlockSpec((BLOCK_ROWS, TOP_K, RBF_NUM), lambda i: (i, 0, 0)),
            pl.BlockSpec((BLOCK_ROWS, TOP_K, 9), lambda i: (i, 0, 0)),
        ],
        out_shape=[
            jax.ShapeDtypeStruct((N, TOP_K), jnp.int32),
            jax.ShapeDtypeStruct((N, TOP_K), jnp.float32),
            jax.ShapeDtypeStruct((N, TOP_K, 3), jnp.float32),
            jax.ShapeDtypeStruct((N, TOP_K, RBF_NUM), jnp.float32),
            jax.ShapeDtypeStruct((N, TOP_K, 9), jnp.float32),
        ],
    )(positions, positions.T, hop_distances, rnd)

    dst, dist, vec, rbf, rot = out
    src = jnp.repeat(jnp.arange(N, dtype=jnp.int32), TOP_K)
    edge_index = jnp.stack([src, dst.reshape(-1)], axis=0)
    edge_distance = dist.reshape(-1)
    edge_distance_vec = vec.reshape(-1, 3)
    edge_distance_rbf = rbf.reshape(-1, RBF_NUM)
    edge_rot_mat = rot.reshape(-1, 3, 3)
    return (edge_index, edge_distance, edge_distance_vec,
            edge_distance_rbf, edge_rot_mat)


# speedup vs baseline: 12.7340x; 2.0355x over previous
import functools

import jax
import jax.numpy as jnp
import numpy as np
from jax.experimental import pallas as pl

N = 1024
TOP_K = 10
KK1 = TOP_K // 2 + 1
KK2 = TOP_K + 1
RBF_START = 0.0
RBF_STOP = 20.0
RBF_NUM = 64
RBF_WIDTH = 1.0

BLOCK_ROWS = 128
NUM_BLOCKS = N // BLOCK_ROWS

_BIG = 1e9


def _graph_kernel(pos_ref, post_ref, hop_ref, rx_ref, ry_ref, rz_ref,
                  dst_ref, dist_ref, vx_ref, vy_ref, vz_ref, rbf_ref,
                  *rot_refs):
    pid = pl.program_id(0)
    row0 = pid * BLOCK_ROWS

    post = post_ref[...]
    hop = hop_ref[...]
    posb = pos_ref[pl.ds(row0, BLOCK_ROWS), :]

    col = jax.lax.broadcasted_iota(jnp.int32, (BLOCK_ROWS, N), 1)
    rowi = jax.lax.broadcasted_iota(jnp.int32, (BLOCK_ROWS, N), 0) + row0

    key = jnp.where(hop == 0.0, 999 * N, hop.astype(jnp.int32) * N) + col
    m1 = jnp.zeros((BLOCK_ROWS, N), dtype=jnp.bool_)
    for _ in range(KK1):
        mn = jnp.min(key, axis=1, keepdims=True)
        sel = key == mn
        m1 = jnp.logical_or(m1, sel)
        key = jnp.where(sel, jnp.int32(2**30), key)
    dmask = jnp.logical_and(m1, hop > 0.0)

    px, py, pz = post[0:1, :], post[1:2, :], post[2:3, :]
    pos = pos_ref[...]
    dots = jax.lax.dot_general(
        posb, pos, (((1,), (1,)), ((), ())),
        preferred_element_type=jnp.float32)
    sa = (posb[:, 0:1] * posb[:, 0:1]
          + posb[:, 1:2] * posb[:, 1:2]
          + posb[:, 2:3] * posb[:, 2:3])
    sb = px * px + py * py + pz * pz
    d2 = sa + sb - 2.0 * dots
    R = jnp.sqrt(jnp.maximum(d2, 0.0))

    work = jnp.where(dmask, 0.0, R)
    m2 = jnp.zeros((BLOCK_ROWS, N), dtype=jnp.bool_)
    for _ in range(KK2):
        mn = jnp.min(work, axis=1, keepdims=True)
        jmin = jnp.min(jnp.where(work == mn, col, N), axis=1, keepdims=True)
        sel = col == jmin
        m2 = jnp.logical_or(m2, sel)
        work = jnp.where(sel, _BIG, work)
    emask = jnp.logical_and(m2, col != rowi)

    widx = jnp.where(emask, col, N)
    dst_cols, dist_cols, vx_cols, vy_cols, vz_cols = [], [], [], [], []
    for _ in range(TOP_K):
        jm = jnp.min(widx, axis=1)
        onehot = col == jm[:, None]
        dst_cols.append(jm[:, None])
        dist_cols.append(
            jnp.sum(jnp.where(onehot, R, 0.0), axis=1)[:, None])
        gx = jnp.sum(jnp.where(onehot, px, 0.0), axis=1)[:, None]
        gy = jnp.sum(jnp.where(onehot, py, 0.0), axis=1)[:, None]
        gz = jnp.sum(jnp.where(onehot, pz, 0.0), axis=1)[:, None]
        vx_cols.append(gx - posb[:, 0:1])
        vy_cols.append(gy - posb[:, 1:2])
        vz_cols.append(gz - posb[:, 2:3])
        widx = jnp.where(onehot, N, widx)

    dst_ref[...] = jnp.concatenate(dst_cols, axis=1)
    dist_ref[...] = jnp.concatenate(dist_cols, axis=1)
    vx_ref[...] = jnp.concatenate(vx_cols, axis=1)
    vy_ref[...] = jnp.concatenate(vy_cols, axis=1)
    vz_ref[...] = jnp.concatenate(vz_cols, axis=1)

    dist = dist_ref[...]
    vx, vy, vz = vx_ref[...], vy_ref[...], vz_ref[...]

    step = (RBF_STOP - RBF_START) / (RBF_NUM - 1)
    offset = (RBF_START
              + step * jax.lax.broadcasted_iota(
                  jnp.int32, (1, 1, RBF_NUM), 2).astype(jnp.float32))
    coeff = -0.5 / (RBF_WIDTH * step) ** 2
    rbf_ref[...] = jnp.exp(coeff * (dist[..., None] - offset) ** 2)

    dx, dy, dz = vx, vy, vz
    dd = jnp.sqrt(jnp.maximum(dx * dx + dy * dy + dz * dz, 1e-12))
    nxx, nxy, nxz = dx / dd, dy / dd, dz / dd

    rx, ry, rz = rx_ref[...], ry_ref[...], rz_ref[...]
    rn = jnp.sqrt(rx * rx + ry * ry + rz * rz)
    ex, ey, ez = rx / rn, ry / rn, rz / rn

    vd_a = jnp.abs(ex * nxx + ey * nxy + ez * nxz)
    vd_b = jnp.abs(-ey * nxx + ex * nxy + ez * nxz)
    take_b = vd_a > vd_b
    ex2 = jnp.where(take_b, -ey, ex)
    ey2 = jnp.where(take_b, ex, ey)
    ez2 = jnp.where(take_b, ez, ez)
    vd = jnp.minimum(vd_a, vd_b)
    vd_c = jnp.abs(ex * nxx - ez * nxy + ey * nxz)
    take_c = vd > vd_c
    ex3 = jnp.where(take_c, ex, ex2)
    ey3 = jnp.where(take_c, -ez, ey2)
    ez3 = jnp.where(take_c, ey, ez2)

    zx = nxy * ez3 - nxz * ey3
    zy = nxz * ex3 - nxx * ez3
    zz = nxx * ey3 - nxy * ex3
    zn = jnp.sqrt(zx * zx + zy * zy + zz * zz)
    zx, zy, zz = zx / zn, zy / zn, zz / zn
    yx = nxy * zz - nxz * zy
    yy = nxz * zx - nxx * zz
    yz = nxx * zy - nxy * zx
    yn = jnp.sqrt(yx * yx + yy * yy + yz * yz)
    yx, yy, yz = yx / yn, yy / yn, yz / yn

    comps = [zx, nxx, -yx, zy, nxy, -yy, zz, nxz, -yz]
    for c, comp in zip(rot_refs, comps):
        c[...] = comp


@functools.partial(jax.jit, static_argnums=())
def kernel(positions, hop_distances):
    rnd = jax.random.uniform(jax.random.key(42), (N * TOP_K, 3),
                             dtype=jnp.float32) - 0.5
    rnd = rnd.reshape(N, TOP_K, 3)

    bk_spec = pl.BlockSpec((BLOCK_ROWS, TOP_K), lambda i: (i, 0))
    bk_shape = jax.ShapeDtypeStruct((N, TOP_K), jnp.float32)
    grid = (NUM_BLOCKS,)
    out = pl.pallas_call(
        _graph_kernel,
        grid=grid,
        in_specs=[
            pl.BlockSpec((N, 3), lambda i: (0, 0)),
            pl.BlockSpec((3, N), lambda i: (0, 0)),
            pl.BlockSpec((BLOCK_ROWS, N), lambda i: (i, 0)),
            bk_spec, bk_spec, bk_spec,
        ],
        out_specs=(
            [bk_spec, bk_spec, bk_spec, bk_spec, bk_spec]
            + [pl.BlockSpec((BLOCK_ROWS, TOP_K, RBF_NUM), lambda i: (i, 0, 0))]
            + [bk_spec] * 9),
        out_shape=(
            [jax.ShapeDtypeStruct((N, TOP_K), jnp.int32),
             bk_shape, bk_shape, bk_shape, bk_shape,
             jax.ShapeDtypeStruct((N, TOP_K, RBF_NUM), jnp.float32)]
            + [bk_shape] * 9),
    )(positions, positions.T, hop_distances,
      rnd[:, :, 0], rnd[:, :, 1], rnd[:, :, 2])

    dst, dist, vx, vy, vz, rbf = out[:6]
    rot9 = out[6:]
    src = jnp.repeat(jnp.arange(N, dtype=jnp.int32), TOP_K)
    edge_index = jnp.stack([src, dst.reshape(-1)], axis=0)
    edge_distance = dist.reshape(-1)
    edge_distance_vec = jnp.stack(
        [vx.reshape(-1), vy.reshape(-1), vz.reshape(-1)], axis=1)
    edge_distance_rbf = rbf.reshape(-1, RBF_NUM)
    edge_rot_mat = jnp.stack(
        [r.reshape(-1) for r in rot9], axis=1).reshape(-1, 3, 3)
    return (edge_index, edge_distance, edge_distance_vec,
            edge_distance_rbf, edge_rot_mat)


# speedup vs baseline: 16.3277x; 1.2822x over previous
import functools

import jax
import jax.numpy as jnp
import numpy as np
from jax.experimental import pallas as pl

N = 1024
TOP_K = 10
KK1 = TOP_K // 2 + 1
KK2 = TOP_K + 1
RBF_START = 0.0
RBF_STOP = 20.0
RBF_NUM = 64
RBF_WIDTH = 1.0

BLOCK_ROWS = 128
NUM_BLOCKS = N // BLOCK_ROWS

_BIG = 1e9


def _graph_kernel(pos_ref, post_ref, hop_ref, rx_ref, ry_ref, rz_ref,
                  dst_ref, dist_ref, vx_ref, vy_ref, vz_ref, rbf_ref,
                  *rot_refs):
    pid = pl.program_id(0)
    row0 = pid * BLOCK_ROWS

    post = post_ref[...]
    hop = hop_ref[...]
    posb = pos_ref[pl.ds(row0, BLOCK_ROWS), :]

    col = jax.lax.broadcasted_iota(jnp.int32, (BLOCK_ROWS, N), 1)
    rowi = jax.lax.broadcasted_iota(jnp.int32, (BLOCK_ROWS, N), 0) + row0

    colf = col.astype(jnp.float32)
    key = jnp.where(hop == 0.0, jnp.float32(999.0), hop) * N + colf
    for _ in range(KK1):
        mn = jnp.min(key, axis=1, keepdims=True)
        key = jnp.where(key == mn, jnp.float32(4e6), key)
    dmask = jnp.logical_and(key == 4e6, hop > 0.0)

    px, py, pz = post[0:1, :], post[1:2, :], post[2:3, :]
    pos = pos_ref[...]
    dots = jax.lax.dot_general(
        posb, pos, (((1,), (1,)), ((), ())),
        preferred_element_type=jnp.float32)
    sa = (posb[:, 0:1] * posb[:, 0:1]
          + posb[:, 1:2] * posb[:, 1:2]
          + posb[:, 2:3] * posb[:, 2:3])
    sb = px * px + py * py + pz * pz
    d2 = sa + sb - 2.0 * dots
    R = jnp.sqrt(jnp.maximum(d2, 0.0))

    work = jnp.where(dmask, 0.0, R)
    for _ in range(KK2):
        mn = jnp.min(work, axis=1, keepdims=True)
        jmin = jnp.min(jnp.where(work == mn, colf, jnp.float32(N)),
                       axis=1, keepdims=True)
        work = jnp.where(colf == jmin, _BIG, work)
    emask = jnp.logical_and(work == _BIG, col != rowi)

    widx = jnp.where(emask, colf, jnp.float32(N))
    dst_cols, dist_cols, vx_cols, vy_cols, vz_cols = [], [], [], [], []
    for _ in range(TOP_K):
        jm = jnp.min(widx, axis=1)
        onehot = colf == jm[:, None]
        dst_cols.append(jm.astype(jnp.int32)[:, None])
        dist_cols.append(
            jnp.sum(jnp.where(onehot, R, 0.0), axis=1)[:, None])
        gx = jnp.sum(jnp.where(onehot, px, 0.0), axis=1)[:, None]
        gy = jnp.sum(jnp.where(onehot, py, 0.0), axis=1)[:, None]
        gz = jnp.sum(jnp.where(onehot, pz, 0.0), axis=1)[:, None]
        vx_cols.append(gx - posb[:, 0:1])
        vy_cols.append(gy - posb[:, 1:2])
        vz_cols.append(gz - posb[:, 2:3])
        widx = jnp.where(onehot, N, widx)

    dst_ref[...] = jnp.concatenate(dst_cols, axis=1)
    dist_ref[...] = jnp.concatenate(dist_cols, axis=1)
    vx_ref[...] = jnp.concatenate(vx_cols, axis=1)
    vy_ref[...] = jnp.concatenate(vy_cols, axis=1)
    vz_ref[...] = jnp.concatenate(vz_cols, axis=1)

    dist = dist_ref[...]
    vx, vy, vz = vx_ref[...], vy_ref[...], vz_ref[...]

    step = (RBF_STOP - RBF_START) / (RBF_NUM - 1)
    offset = (RBF_START
              + step * jax.lax.broadcasted_iota(
                  jnp.int32, (1, 1, RBF_NUM), 2).astype(jnp.float32))
    coeff = -0.5 / (RBF_WIDTH * step) ** 2
    rbf_ref[...] = jnp.exp(coeff * (dist[..., None] - offset) ** 2)

    dx, dy, dz = vx, vy, vz
    dd = jnp.sqrt(jnp.maximum(dx * dx + dy * dy + dz * dz, 1e-12))
    nxx, nxy, nxz = dx / dd, dy / dd, dz / dd

    rx, ry, rz = rx_ref[...], ry_ref[...], rz_ref[...]
    rn = jnp.sqrt(rx * rx + ry * ry + rz * rz)
    ex, ey, ez = rx / rn, ry / rn, rz / rn

    vd_a = jnp.abs(ex * nxx + ey * nxy + ez * nxz)
    vd_b = jnp.abs(-ey * nxx + ex * nxy + ez * nxz)
    take_b = vd_a > vd_b
    ex2 = jnp.where(take_b, -ey, ex)
    ey2 = jnp.where(take_b, ex, ey)
    ez2 = jnp.where(take_b, ez, ez)
    vd = jnp.minimum(vd_a, vd_b)
    vd_c = jnp.abs(ex * nxx - ez * nxy + ey * nxz)
    take_c = vd > vd_c
    ex3 = jnp.where(take_c, ex, ex2)
    ey3 = jnp.where(take_c, -ez, ey2)
    ez3 = jnp.where(take_c, ey, ez2)

    zx = nxy * ez3 - nxz * ey3
    zy = nxz * ex3 - nxx * ez3
    zz = nxx * ey3 - nxy * ex3
    zn = jnp.sqrt(zx * zx + zy * zy + zz * zz)
    zx, zy, zz = zx / zn, zy / zn, zz / zn
    yx = nxy * zz - nxz * zy
    yy = nxz * zx - nxx * zz
    yz = nxx * zy - nxy * zx
    yn = jnp.sqrt(yx * yx + yy * yy + yz * yz)
    yx, yy, yz = yx / yn, yy / yn, yz / yn

    comps = [zx, nxx, -yx, zy, nxy, -yy, zz, nxz, -yz]
    for c, comp in zip(rot_refs, comps):
        c[...] = comp


@functools.partial(jax.jit, static_argnums=())
def kernel(positions, hop_distances):
    rnd = jax.random.uniform(jax.random.key(42), (N * TOP_K, 3),
                             dtype=jnp.float32) - 0.5
    rnd = rnd.reshape(N, TOP_K, 3)

    bk_spec = pl.BlockSpec((BLOCK_ROWS, TOP_K), lambda i: (i, 0))
    bk_shape = jax.ShapeDtypeStruct((N, TOP_K), jnp.float32)
    grid = (NUM_BLOCKS,)
    out = pl.pallas_call(
        _graph_kernel,
        grid=grid,
        in_specs=[
            pl.BlockSpec((N, 3), lambda i: (0, 0)),
            pl.BlockSpec((3, N), lambda i: (0, 0)),
            pl.BlockSpec((BLOCK_ROWS, N), lambda i: (i, 0)),
            bk_spec, bk_spec, bk_spec,
        ],
        out_specs=(
            [bk_spec, bk_spec, bk_spec, bk_spec, bk_spec]
            + [pl.BlockSpec((BLOCK_ROWS, TOP_K, RBF_NUM), lambda i: (i, 0, 0))]
            + [bk_spec] * 9),
        out_shape=(
            [jax.ShapeDtypeStruct((N, TOP_K), jnp.int32),
             bk_shape, bk_shape, bk_shape, bk_shape,
             jax.ShapeDtypeStruct((N, TOP_K, RBF_NUM), jnp.float32)]
            + [bk_shape] * 9),
    )(positions, positions.T, hop_distances,
      rnd[:, :, 0], rnd[:, :, 1], rnd[:, :, 2])

    dst, dist, vx, vy, vz, rbf = out[:6]
    rot9 = out[6:]
    src = jnp.repeat(jnp.arange(N, dtype=jnp.int32), TOP_K)
    edge_index = jnp.stack([src, dst.reshape(-1)], axis=0)
    edge_distance = dist.reshape(-1)
    edge_distance_vec = jnp.stack(
        [vx.reshape(-1), vy.reshape(-1), vz.reshape(-1)], axis=1)
    edge_distance_rbf = rbf.reshape(-1, RBF_NUM)
    edge_rot_mat = jnp.stack(
        [r.reshape(-1) for r in rot9], axis=1).reshape(-1, 3, 3)
    return (edge_index, edge_distance, edge_distance_vec,
            edge_distance_rbf, edge_rot_mat)


# speedup vs baseline: 17.4303x; 1.0675x over previous
import functools

import jax
import jax.numpy as jnp
import numpy as np
from jax.experimental import pallas as pl

N = 1024
TOP_K = 10
KK1 = TOP_K // 2 + 1
KK2 = TOP_K + 1
RBF_START = 0.0
RBF_STOP = 20.0
RBF_NUM = 64
RBF_WIDTH = 1.0

BLOCK_ROWS = 256
NUM_BLOCKS = N // BLOCK_ROWS

_BIG = 1e9

_RND = np.asarray(
    jax.random.uniform(jax.random.key(42), (N * TOP_K, 3),
                       dtype=jnp.float32)) - np.float32(0.5)
_RX = _RND[:, 0].reshape(N, TOP_K)
_RY = _RND[:, 1].reshape(N, TOP_K)
_RZ = _RND[:, 2].reshape(N, TOP_K)


def _graph_kernel(pos_ref, post_ref, hop_ref, rx_ref, ry_ref, rz_ref,
                  dst_ref, dist_ref, vx_ref, vy_ref, vz_ref, rbf_ref,
                  *rot_refs):
    pid = pl.program_id(0)
    row0 = pid * BLOCK_ROWS

    post = post_ref[...]
    hop = hop_ref[...]
    posb = pos_ref[pl.ds(row0, BLOCK_ROWS), :]

    col = jax.lax.broadcasted_iota(jnp.int32, (BLOCK_ROWS, N), 1)
    rowi = jax.lax.broadcasted_iota(jnp.int32, (BLOCK_ROWS, N), 0) + row0

    colf = col.astype(jnp.float32)
    key = jnp.where(hop == 0.0, jnp.float32(999.0), hop) * N + colf
    for _ in range(KK1):
        mn = jnp.min(key, axis=1, keepdims=True)
        key = jnp.where(key == mn, jnp.float32(4e6), key)
    dmask = jnp.logical_and(key == 4e6, hop > 0.0)

    px, py, pz = post[0:1, :], post[1:2, :], post[2:3, :]
    pos = pos_ref[...]
    dots = jax.lax.dot_general(
        posb, pos, (((1,), (1,)), ((), ())),
        preferred_element_type=jnp.float32)
    sa = (posb[:, 0:1] * posb[:, 0:1]
          + posb[:, 1:2] * posb[:, 1:2]
          + posb[:, 2:3] * posb[:, 2:3])
    sb = px * px + py * py + pz * pz
    d2 = sa + sb - 2.0 * dots
    R = jnp.sqrt(jnp.maximum(d2, 0.0))

    work = jnp.where(dmask, 0.0, R)
    for _ in range(KK2):
        mn = jnp.min(work, axis=1, keepdims=True)
        jmin = jnp.min(jnp.where(work == mn, colf, jnp.float32(N)),
                       axis=1, keepdims=True)
        work = jnp.where(colf == jmin, _BIG, work)
    emask = jnp.logical_and(work == _BIG, col != rowi)

    widx = jnp.where(emask, colf, jnp.float32(N))
    dst_cols, dist_cols, vx_cols, vy_cols, vz_cols = [], [], [], [], []
    for _ in range(TOP_K):
        jm = jnp.min(widx, axis=1)
        onehot = colf == jm[:, None]
        dst_cols.append(jm.astype(jnp.int32)[:, None])
        dist_cols.append(
            jnp.sum(jnp.where(onehot, R, 0.0), axis=1)[:, None])
        gx = jnp.sum(jnp.where(onehot, px, 0.0), axis=1)[:, None]
        gy = jnp.sum(jnp.where(onehot, py, 0.0), axis=1)[:, None]
        gz = jnp.sum(jnp.where(onehot, pz, 0.0), axis=1)[:, None]
        vx_cols.append(gx - posb[:, 0:1])
        vy_cols.append(gy - posb[:, 1:2])
        vz_cols.append(gz - posb[:, 2:3])
        widx = jnp.where(onehot, N, widx)

    dst_ref[...] = jnp.concatenate(dst_cols, axis=1)
    dist_ref[...] = jnp.concatenate(dist_cols, axis=1)
    vx_ref[...] = jnp.concatenate(vx_cols, axis=1)
    vy_ref[...] = jnp.concatenate(vy_cols, axis=1)
    vz_ref[...] = jnp.concatenate(vz_cols, axis=1)

    dist = dist_ref[...]
    vx, vy, vz = vx_ref[...], vy_ref[...], vz_ref[...]

    step = (RBF_STOP - RBF_START) / (RBF_NUM - 1)
    offset = (RBF_START
              + step * jax.lax.broadcasted_iota(
                  jnp.int32, (1, 1, RBF_NUM), 2).astype(jnp.float32))
    coeff = -0.5 / (RBF_WIDTH * step) ** 2
    rbf_ref[...] = jnp.exp(coeff * (dist[..., None] - offset) ** 2)

    dx, dy, dz = vx, vy, vz
    dd = jnp.sqrt(jnp.maximum(dx * dx + dy * dy + dz * dz, 1e-12))
    nxx, nxy, nxz = dx / dd, dy / dd, dz / dd

    rx, ry, rz = rx_ref[...], ry_ref[...], rz_ref[...]
    rn = jnp.sqrt(rx * rx + ry * ry + rz * rz)
    ex, ey, ez = rx / rn, ry / rn, rz / rn

    vd_a = jnp.abs(ex * nxx + ey * nxy + ez * nxz)
    vd_b = jnp.abs(-ey * nxx + ex * nxy + ez * nxz)
    take_b = vd_a > vd_b
    ex2 = jnp.where(take_b, -ey, ex)
    ey2 = jnp.where(take_b, ex, ey)
    ez2 = jnp.where(take_b, ez, ez)
    vd = jnp.minimum(vd_a, vd_b)
    vd_c = jnp.abs(ex * nxx - ez * nxy + ey * nxz)
    take_c = vd > vd_c
    ex3 = jnp.where(take_c, ex, ex2)
    ey3 = jnp.where(take_c, -ez, ey2)
    ez3 = jnp.where(take_c, ey, ez2)

    zx = nxy * ez3 - nxz * ey3
    zy = nxz * ex3 - nxx * ez3
    zz = nxx * ey3 - nxy * ex3
    zn = jnp.sqrt(zx * zx + zy * zy + zz * zz)
    zx, zy, zz = zx / zn, zy / zn, zz / zn
    yx = nxy * zz - nxz * zy
    yy = nxz * zx - nxx * zz
    yz = nxx * zy - nxy * zx
    yn = jnp.sqrt(yx * yx + yy * yy + yz * yz)
    yx, yy, yz = yx / yn, yy / yn, yz / yn

    comps = [zx, nxx, -yx, zy, nxy, -yy, zz, nxz, -yz]
    for c, comp in zip(rot_refs, comps):
        c[...] = comp


@functools.partial(jax.jit, static_argnums=())
def kernel(positions, hop_distances):
    bk_spec = pl.BlockSpec((BLOCK_ROWS, TOP_K), lambda i: (i, 0))
    bk_shape = jax.ShapeDtypeStruct((N, TOP_K), jnp.float32)
    grid = (NUM_BLOCKS,)
    out = pl.pallas_call(
        _graph_kernel,
        grid=grid,
        in_specs=[
            pl.BlockSpec((N, 3), lambda i: (0, 0)),
            pl.BlockSpec((3, N), lambda i: (0, 0)),
            pl.BlockSpec((BLOCK_ROWS, N), lambda i: (i, 0)),
            bk_spec, bk_spec, bk_spec,
        ],
        out_specs=(
            [bk_spec, bk_spec, bk_spec, bk_spec, bk_spec]
            + [pl.BlockSpec((BLOCK_ROWS, TOP_K, RBF_NUM), lambda i: (i, 0, 0))]
            + [bk_spec] * 9),
        out_shape=(
            [jax.ShapeDtypeStruct((N, TOP_K), jnp.int32),
             bk_shape, bk_shape, bk_shape, bk_shape,
             jax.ShapeDtypeStruct((N, TOP_K, RBF_NUM), jnp.float32)]
            + [bk_shape] * 9),
    )(positions, positions.T, hop_distances, _RX, _RY, _RZ)

    dst, dist, vx, vy, vz, rbf = out[:6]
    rot9 = out[6:]
    src = jnp.repeat(jnp.arange(N, dtype=jnp.int32), TOP_K)
    edge_index = jnp.stack([src, dst.reshape(-1)], axis=0)
    edge_distance = dist.reshape(-1)
    edge_distance_vec = jnp.stack(
        [vx.reshape(-1), vy.reshape(-1), vz.reshape(-1)], axis=1)
    edge_distance_rbf = rbf.reshape(-1, RBF_NUM)
    edge_rot_mat = jnp.stack(
        [r.reshape(-1) for r in rot9], axis=1).reshape(-1, 3, 3)
    return (edge_index, edge_distance, edge_distance_vec,
            edge_distance_rbf, edge_rot_mat)
